# baseline (speedup 1.0000x reference)
import functools

import jax
import jax.numpy as jnp
from jax import lax
from jax.experimental import pallas as pl
from jax.experimental.pallas import tpu as pltpu
from jax.experimental.pallas import tpu_sc as plsc

_NC = 2
_NS = 16
_NW = _NC * _NS
_K = 128
_NB = 8
_L = 16


@functools.lru_cache(maxsize=None)
def _deg_kernel(n_pad: int, epw: int):
    mesh = plsc.VectorSubcoreMesh(core_axis_name="c", subcore_axis_name="s")
    acc_len = _NB * n_pad

    @functools.partial(
        pl.kernel,
        out_type=jax.ShapeDtypeStruct((_NW * acc_len,), jnp.float32),
        mesh=mesh,
        scratch_types=[
            pltpu.VMEM((epw,), jnp.int32),
            pltpu.VMEM((acc_len,), jnp.float32),
        ],
        compiler_params=pltpu.CompilerParams(needs_layout_passes=False),
    )
    def deg(dst_hbm, out_hbm, dst_v, acc):
        cid = lax.axis_index("c")
        sid = lax.axis_index("s")
        wid = cid * _NS + sid
        pltpu.sync_copy(dst_hbm.at[pl.ds(wid * epw, epw)], dst_v)

        zeros = jnp.zeros((_L,), jnp.float32)

        @pl.loop(0, acc_len // _L)
        def _(i):
            acc[pl.ds(i * _L, _L)] = zeros

        ones = jnp.ones((_L,), jnp.float32)
        lane = lax.iota(jnp.int32, _L)
        bank_off = lax.bitwise_and(lane, _NB - 1) * n_pad
        mlow = lane < _NB
        mhigh = lane >= _NB

        @pl.loop(0, epw // _L)
        def _(c):
            idx = dst_v[pl.ds(c * _L, _L)] + bank_off
            plsc.addupdate_scatter(acc, [idx], ones, mask=mlow)
            plsc.addupdate_scatter(acc, [idx], ones, mask=mhigh)

        pltpu.sync_copy(acc, out_hbm.at[pl.ds(wid * acc_len, acc_len)])

    return deg


@functools.lru_cache(maxsize=None)
def _prop_kernel(n_pad: int, width: int, nchunk: int):
    half = n_pad // 2
    rows_per_tile = half // _NS
    mesh = plsc.VectorSubcoreMesh(core_axis_name="c", subcore_axis_name="s")

    @functools.partial(
        pl.kernel,
        out_type=jax.ShapeDtypeStruct((n_pad, width), jnp.float32),
        mesh=mesh,
        scratch_types=[
            pltpu.VMEM((nchunk, _K), jnp.int32),
            pltpu.VMEM((nchunk, _K), jnp.int32),
            pltpu.VMEM((_K, width), jnp.float32),
            pltpu.VMEM((_K, width), jnp.float32),
            pltpu.VMEM_SHARED((half, width), jnp.float32),
            pltpu.SemaphoreType.DMA,
            pltpu.SemaphoreType.DMA,
        ],
        compiler_params=pltpu.CompilerParams(use_tc_tiling_on_sc=False),
    )
    def prop(table_hbm, src_hbm, dst_hbm, out_hbm,
             src_v, dst_v, buf0, buf1, acc, sem0, sem1):
        cid = lax.axis_index("c")
        sid = lax.axis_index("s")
        shard = (cid * _NS + sid) * nchunk
        pltpu.sync_copy(src_hbm.at[pl.ds(shard, nchunk)], src_v)
        pltpu.sync_copy(dst_hbm.at[pl.ds(shard, nchunk)], dst_v)
        rbase = sid * rows_per_tile
        pltpu.sync_copy(
            table_hbm.at[pl.ds(cid * half + rbase, rows_per_tile)],
            acc.at[pl.ds(rbase, rows_per_tile)])
        plsc.subcore_barrier()

        def gather(c, buf, sem):
            pltpu.async_copy(table_hbm.at[src_v.at[c]], buf, sem)

        def scat(c, buf):
            pltpu.sync_copy(
                buf, acc.at[plsc.Indices(dst_v.at[c], ignored_value=-1)],
                add=True)

        gather(0, buf0, sem0)
        gather(1, buf1, sem1)

        @pl.loop(0, nchunk, step=2)
        def _(c):
            pltpu.make_async_copy(table_hbm.at[src_v.at[c]], buf0, sem0).wait()
            scat(c, buf0)

            @pl.when(c + 2 < nchunk)
            def _():
                gather(c + 2, buf0, sem0)

            pltpu.make_async_copy(
                table_hbm.at[src_v.at[c + 1]], buf1, sem1).wait()
            scat(c + 1, buf1)

            @pl.when(c + 3 < nchunk)
            def _():
                gather(c + 3, buf1, sem1)

        plsc.subcore_barrier()
        pltpu.sync_copy(acc.at[pl.ds(rbase, rows_per_tile)],
                        out_hbm.at[pl.ds(cid * half + rbase, rows_per_tile)])

    return prop


def _tc_prep_body(degp_ref, src_ref, dst_ref, degrow_ref, srcm_ref, dstm_ref,
                  *, n_pad, rows):
    degrow_ref[...] = jnp.sum(degp_ref[...], axis=0, keepdims=True)
    src = src_ref[...]
    dst = dst_ref[...]
    half = n_pad // 2
    dummy = jnp.int32(n_pad - 1)
    sent = jnp.int32(-1)
    lo = dst < half
    srcm_ref[:rows, :] = jnp.where(lo, src, dummy)
    dstm_ref[:rows, :] = jnp.where(lo, dst, sent)
    srcm_ref[rows:, :] = jnp.where(lo, dummy, src)
    dstm_ref[rows:, :] = jnp.where(lo, sent, dst - half)


def _tc_scale_body(deg_ref, x_ref, dinv_ref, xs_ref):
    dinv = lax.rsqrt(deg_ref[...] + 1.0)
    dinv_ref[...] = dinv
    xs_ref[...] = x_ref[...] * dinv


def _tc_mid_body(y_ref, dinv_ref, w1_ref, b1_ref, w2_ref, g_ref):
    z = y_ref[...] * dinv_ref[...]
    h = jnp.dot(z, w1_ref[...], preferred_element_type=jnp.float32)
    h = jnp.maximum(h + b1_ref[...], 0.0)
    g = jnp.dot(h, w2_ref[...], preferred_element_type=jnp.float32)
    g_ref[...] = g * dinv_ref[...]


def _tc_out_body(y_ref, dinv_ref, b2_ref, out_ref):
    out_ref[...] = y_ref[...] * dinv_ref[...] + b2_ref[...]


def kernel(x, adj, W1, b1, W2, b2):
    n, nfeat = x.shape
    nhid = W2.shape[1]
    e = adj.shape[1]

    n_pad = (n // 256 + 1) * 256
    per_tile = -(-e // (_NS * 16 * _K)) * (16 * _K)
    e_pad = per_tile * _NS
    nchunk = per_tile // _K
    rows = _NS * nchunk

    adj = adj.astype(jnp.int32)
    adj = jnp.pad(adj, ((0, 0), (0, e_pad - e)), constant_values=n_pad - 1)
    src2d = adj[0].reshape(rows, _K)
    dst2d = adj[1].reshape(rows, _K)
    x_pad = jnp.pad(x, ((0, n_pad - n), (0, 0)))

    epw = e_pad // _NW
    degp = _deg_kernel(n_pad, epw)(adj[1])
    degp = degp.reshape(_NW * _NB, n_pad)

    degrow, src_cat, dst_cat = pl.pallas_call(
        functools.partial(_tc_prep_body, n_pad=n_pad, rows=rows),
        out_shape=[
            jax.ShapeDtypeStruct((1, n_pad), jnp.float32),
            jax.ShapeDtypeStruct((2 * rows, _K), jnp.int32),
            jax.ShapeDtypeStruct((2 * rows, _K), jnp.int32),
        ],
    )(degp, src2d, dst2d)
    deg_col = degrow.reshape(n_pad, 1)

    dinv, xs = pl.pallas_call(
        _tc_scale_body,
        out_shape=[
            jax.ShapeDtypeStruct((n_pad, 1), jnp.float32),
            jax.ShapeDtypeStruct((n_pad, nfeat), jnp.float32),
        ],
    )(deg_col, x_pad)

    y1 = _prop_kernel(n_pad, nfeat, nchunk)(xs, src_cat, dst_cat)

    g = pl.pallas_call(
        _tc_mid_body,
        out_shape=jax.ShapeDtypeStruct((n_pad, nhid), jnp.float32),
    )(y1, dinv, W1, b1.reshape(1, -1), W2)

    y2 = _prop_kernel(n_pad, nhid, nchunk)(g, src_cat, dst_cat)

    out = pl.pallas_call(
        _tc_out_body,
        out_shape=jax.ShapeDtypeStruct((n_pad, nhid), jnp.float32),
    )(y2, dinv, b2.reshape(1, -1))

    return out[:n]


# speedup vs baseline: 1.0001x; 1.0001x over previous
import functools

import jax
import jax.numpy as jnp
from jax import lax
from jax.experimental import pallas as pl
from jax.experimental.pallas import tpu as pltpu
from jax.experimental.pallas import tpu_sc as plsc

_NC = 2
_NS = 16
_NW = _NC * _NS
_K = 128
_NB = 8
_L = 16


@functools.lru_cache(maxsize=None)
def _deg_kernel(n_pad: int, epw: int):
    mesh = plsc.VectorSubcoreMesh(core_axis_name="c", subcore_axis_name="s")
    acc_len = _NB * n_pad

    @functools.partial(
        pl.kernel,
        out_type=jax.ShapeDtypeStruct((_NW * acc_len,), jnp.float32),
        mesh=mesh,
        scratch_types=[
            pltpu.VMEM((epw,), jnp.int32),
            pltpu.VMEM((acc_len,), jnp.float32),
        ],
        compiler_params=pltpu.CompilerParams(needs_layout_passes=False),
    )
    def deg(dst_hbm, out_hbm, dst_v, acc):
        cid = lax.axis_index("c")
        sid = lax.axis_index("s")
        wid = cid * _NS + sid
        pltpu.sync_copy(dst_hbm.at[pl.ds(wid * epw, epw)], dst_v)

        zeros = jnp.zeros((_L,), jnp.float32)

        @pl.loop(0, acc_len // _L)
        def _(i):
            acc[pl.ds(i * _L, _L)] = zeros

        ones = jnp.ones((_L,), jnp.float32)
        lane = lax.iota(jnp.int32, _L)
        bank_off = lax.bitwise_and(lane, _NB - 1) * n_pad
        mlow = lane < _NB
        mhigh = lane >= _NB

        @pl.loop(0, epw // _L)
        def _(c):
            idx = dst_v[pl.ds(c * _L, _L)] + bank_off
            plsc.addupdate_scatter(acc, [idx], ones, mask=mlow)
            plsc.addupdate_scatter(acc, [idx], ones, mask=mhigh)

        pltpu.sync_copy(acc, out_hbm.at[pl.ds(wid * acc_len, acc_len)])

    return deg


@functools.lru_cache(maxsize=None)
def _prop_kernel(n_pad: int, width: int, nchunk: int):
    half = n_pad // 2
    rows_per_tile = half // _NS
    mesh = plsc.VectorSubcoreMesh(core_axis_name="c", subcore_axis_name="s")

    @functools.partial(
        pl.kernel,
        out_type=jax.ShapeDtypeStruct((n_pad, width), jnp.float32),
        mesh=mesh,
        scratch_types=[
            pltpu.VMEM((8 * _K,), jnp.int32),
            pltpu.VMEM((8, _K), jnp.int32),
            pltpu.VMEM((4 * _K, width), jnp.float32),
            pltpu.VMEM_SHARED((half, width), jnp.float32),
            pltpu.SemaphoreType.DMA,
        ],
        compiler_params=pltpu.CompilerParams(use_tc_tiling_on_sc=False),
    )
    def prop(table_hbm, srcf_hbm, dst_hbm, out_hbm,
             src_v, dst_v, bigbuf, acc, sem0):
        cid = lax.axis_index("c")
        sid = lax.axis_index("s")
        shard = (cid * _NS + sid) * nchunk
        rbase = sid * rows_per_tile
        pltpu.sync_copy(
            table_hbm.at[pl.ds(cid * half + rbase, rows_per_tile)],
            acc.at[pl.ds(rbase, rows_per_tile)])
        plsc.subcore_barrier()

        def scat(c, buf):
            pltpu.sync_copy(
                buf, acc.at[plsc.Indices(dst_v.at[c], ignored_value=-1)],
                add=True)

        @pl.loop(0, nchunk // 8)
        def _(g):
            row = shard + g * 8
            pltpu.sync_copy(srcf_hbm.at[pl.ds(row * _K, 8 * _K)], src_v)
            pltpu.sync_copy(dst_hbm.at[pl.ds(row, 8)], dst_v)
            for hf in range(2):
                for b in range(4):
                    pltpu.async_copy(
                        table_hbm.at[src_v.at[pl.ds((4 * hf + b) * _K, _K)]],
                        bigbuf.at[pl.ds(b * _K, _K)], sem0)
                pltpu.make_async_copy(
                    table_hbm.at[pl.ds(0, 4 * _K)], bigbuf, sem0).wait()
                for b in range(4):
                    scat(4 * hf + b, bigbuf.at[pl.ds(b * _K, _K)])

        plsc.subcore_barrier()
        pltpu.sync_copy(acc.at[pl.ds(rbase, rows_per_tile)],
                        out_hbm.at[pl.ds(cid * half + rbase, rows_per_tile)])

    return prop


def _tc_prep_body(degp_ref, src_ref, dst_ref, degrow_ref, srcm_ref, dstm_ref,
                  *, n_pad, rows):
    degrow_ref[...] = jnp.sum(degp_ref[...], axis=0, keepdims=True)
    src = src_ref[...]
    dst = dst_ref[...]
    half = n_pad // 2
    dummy = jnp.int32(n_pad - 1)
    sent = jnp.int32(-1)
    lo = dst < half
    srcm_ref[:rows, :] = jnp.where(lo, src, dummy)
    dstm_ref[:rows, :] = jnp.where(lo, dst, sent)
    srcm_ref[rows:, :] = jnp.where(lo, dummy, src)
    dstm_ref[rows:, :] = jnp.where(lo, sent, dst - half)


def _tc_scale_body(deg_ref, x_ref, dinv_ref, xs_ref):
    dinv = lax.rsqrt(deg_ref[...] + 1.0)
    dinv_ref[...] = dinv
    xs_ref[...] = x_ref[...] * dinv


def _tc_mid_body(y_ref, dinv_ref, w1_ref, b1_ref, w2_ref, g_ref):
    z = y_ref[...] * dinv_ref[...]
    h = jnp.dot(z, w1_ref[...], preferred_element_type=jnp.float32)
    h = jnp.maximum(h + b1_ref[...], 0.0)
    g = jnp.dot(h, w2_ref[...], preferred_element_type=jnp.float32)
    g_ref[...] = g * dinv_ref[...]


def _tc_out_body(y_ref, dinv_ref, b2_ref, out_ref):
    out_ref[...] = y_ref[...] * dinv_ref[...] + b2_ref[...]


def kernel(x, adj, W1, b1, W2, b2):
    n, nfeat = x.shape
    nhid = W2.shape[1]
    e = adj.shape[1]

    n_pad = (n // 256 + 1) * 256
    per_tile = -(-e // (_NS * 16 * _K)) * (16 * _K)
    e_pad = per_tile * _NS
    nchunk = per_tile // _K
    rows = _NS * nchunk

    adj = adj.astype(jnp.int32)
    adj = jnp.pad(adj, ((0, 0), (0, e_pad - e)), constant_values=n_pad - 1)
    src2d = adj[0].reshape(rows, _K)
    dst2d = adj[1].reshape(rows, _K)
    x_pad = jnp.pad(x, ((0, n_pad - n), (0, 0)))

    epw = e_pad // _NW
    degp = _deg_kernel(n_pad, epw)(adj[1])
    degp = degp.reshape(_NW * _NB, n_pad)

    degrow, src_cat, dst_cat = pl.pallas_call(
        functools.partial(_tc_prep_body, n_pad=n_pad, rows=rows),
        out_shape=[
            jax.ShapeDtypeStruct((1, n_pad), jnp.float32),
            jax.ShapeDtypeStruct((2 * rows, _K), jnp.int32),
            jax.ShapeDtypeStruct((2 * rows, _K), jnp.int32),
        ],
    )(degp, src2d, dst2d)
    deg_col = degrow.reshape(n_pad, 1)

    dinv, xs = pl.pallas_call(
        _tc_scale_body,
        out_shape=[
            jax.ShapeDtypeStruct((n_pad, 1), jnp.float32),
            jax.ShapeDtypeStruct((n_pad, nfeat), jnp.float32),
        ],
    )(deg_col, x_pad)

    src_flat = src_cat.reshape(-1)
    y1 = _prop_kernel(n_pad, nfeat, nchunk)(xs, src_flat, dst_cat)

    g = pl.pallas_call(
        _tc_mid_body,
        out_shape=jax.ShapeDtypeStruct((n_pad, nhid), jnp.float32),
    )(y1, dinv, W1, b1.reshape(1, -1), W2)

    y2 = _prop_kernel(n_pad, nhid, nchunk)(g, src_flat, dst_cat)

    out = pl.pallas_call(
        _tc_out_body,
        out_shape=jax.ShapeDtypeStruct((n_pad, nhid), jnp.float32),
    )(y2, dinv, b2.reshape(1, -1))

    return out[:n]


# speedup vs baseline: 10.2461x; 10.2453x over previous
import functools

import jax
import jax.numpy as jnp
from jax import lax
from jax.experimental import pallas as pl
from jax.experimental.pallas import tpu as pltpu
from jax.experimental.pallas import tpu_sc as plsc

_NC = 2
_NS = 16
_NW = _NC * _NS
_CW = 4
_NB = 8
_L = 16
_EC = 8192


@functools.lru_cache(maxsize=None)
def _deg_kernel(n_pad: int, epw: int):
    mesh = plsc.VectorSubcoreMesh(core_axis_name="c", subcore_axis_name="s")
    acc_len = _NB * n_pad

    @functools.partial(
        pl.kernel,
        out_type=jax.ShapeDtypeStruct((_NW * acc_len,), jnp.float32),
        mesh=mesh,
        scratch_types=[
            pltpu.VMEM((epw,), jnp.int32),
            pltpu.VMEM((acc_len,), jnp.float32),
        ],
        compiler_params=pltpu.CompilerParams(needs_layout_passes=False),
    )
    def deg(dst_hbm, out_hbm, dst_v, acc):
        cid = lax.axis_index("c")
        sid = lax.axis_index("s")
        wid = cid * _NS + sid
        pltpu.sync_copy(dst_hbm.at[pl.ds(wid * epw, epw)], dst_v)

        zeros = jnp.zeros((_L,), jnp.float32)

        @pl.loop(0, acc_len // _L)
        def _(i):
            acc[pl.ds(i * _L, _L)] = zeros

        ones = jnp.ones((_L,), jnp.float32)
        lane = lax.iota(jnp.int32, _L)
        bank_off = lax.bitwise_and(lane, _NB - 1) * n_pad
        mlow = lane < _NB
        mhigh = lane >= _NB

        @pl.loop(0, epw // _L)
        def _(c):
            idx = dst_v[pl.ds(c * _L, _L)] + bank_off
            plsc.addupdate_scatter(acc, [idx], ones, mask=mlow)
            plsc.addupdate_scatter(acc, [idx], ones, mask=mhigh)

        pltpu.sync_copy(acc, out_hbm.at[pl.ds(wid * acc_len, acc_len)])

    return deg


@functools.lru_cache(maxsize=None)
def _prop_kernel(n_pad: int, nchunk: int):
    blk = n_pad * _CW
    mesh = plsc.VectorSubcoreMesh(core_axis_name="c", subcore_axis_name="s")

    @functools.partial(
        pl.kernel,
        out_type=jax.ShapeDtypeStruct((_NW * blk,), jnp.float32),
        mesh=mesh,
        scratch_types=[
            pltpu.VMEM((blk,), jnp.float32),
            pltpu.VMEM((blk,), jnp.float32),
            pltpu.VMEM((_EC,), jnp.int32),
            pltpu.VMEM((_EC,), jnp.int32),
        ],
        compiler_params=pltpu.CompilerParams(needs_layout_passes=False),
    )
    def prop(tcb_hbm, src_hbm, dst_hbm, out_hbm, tbl, acc, src_v, dst_v):
        cid = lax.axis_index("c")
        sid = lax.axis_index("s")
        base = (cid * _NS + sid) * blk
        pltpu.sync_copy(tcb_hbm.at[pl.ds(base, blk)], tbl)
        pltpu.sync_copy(tcb_hbm.at[pl.ds(base, blk)], acc)

        lane = lax.iota(jnp.int32, _L)
        colpat = lax.bitwise_and(lane, _CW - 1)
        grp = lax.shift_right_logical(lane, 2)
        emasks = [grp == q for q in range(4)]
        reps = [grp + 4 * q for q in range(4)]

        @pl.loop(0, nchunk)
        def _(ch):
            ebase = ch * _EC
            pltpu.sync_copy(src_hbm.at[pl.ds(ebase, _EC)], src_v)
            pltpu.sync_copy(dst_hbm.at[pl.ds(ebase, _EC)], dst_v)

            @pl.loop(0, _EC // _L)
            def _(i):
                srcv = src_v[pl.ds(i * _L, _L)]
                dstv = dst_v[pl.ds(i * _L, _L)]
                for q in range(4):
                    s_rep = jnp.take_along_axis(srcv, reps[q], axis=0)
                    d_rep = jnp.take_along_axis(dstv, reps[q], axis=0)
                    fs = lax.bitwise_or(lax.shift_left(s_rep, 2), colpat)
                    fd = lax.bitwise_or(lax.shift_left(d_rep, 2), colpat)
                    vals = plsc.load_gather(tbl, [fs])
                    for m in range(4):
                        plsc.addupdate_scatter(acc, [fd], vals,
                                               mask=emasks[m])

        pltpu.sync_copy(acc, out_hbm.at[pl.ds(base, blk)])

    return prop


def _tc_reduce_body(degp_ref, degrow_ref):
    degrow_ref[...] = jnp.sum(degp_ref[...], axis=0, keepdims=True)


def _tc_scale_body(deg_ref, x_ref, dinv_ref, xs_ref):
    dinv = lax.rsqrt(deg_ref[...] + 1.0)
    dinv_ref[...] = dinv
    xs_ref[...] = x_ref[...] * dinv


def _tc_mid_body(y_ref, dinv_ref, w1_ref, b1_ref, w2_ref, g_ref):
    z = y_ref[...] * dinv_ref[...]
    h = jnp.dot(z, w1_ref[...], preferred_element_type=jnp.float32)
    h = jnp.maximum(h + b1_ref[...], 0.0)
    g = jnp.dot(h, w2_ref[...], preferred_element_type=jnp.float32)
    g_ref[...] = g * dinv_ref[...]


def _tc_out_body(y_ref, dinv_ref, b2_ref, out_ref):
    out_ref[...] = y_ref[...] * dinv_ref[...] + b2_ref[...]


def _to_cb(a, n_pad):
    return a.reshape(n_pad, _NW, _CW).transpose(1, 0, 2).reshape(-1)


def _from_cb(a, n_pad):
    return a.reshape(_NW, n_pad, _CW).transpose(1, 0, 2).reshape(n_pad, -1)


def kernel(x, adj, W1, b1, W2, b2):
    n, nfeat = x.shape
    nhid = W2.shape[1]
    e = adj.shape[1]

    n_pad = (n // 256 + 1) * 256
    e_pad = -(-e // _EC) * _EC
    nchunk = e_pad // _EC
    epw = e_pad // _NW

    adj = jnp.pad(adj.astype(jnp.int32), ((0, 0), (0, e_pad - e)),
                  constant_values=n_pad - 1)
    srcf = adj[0]
    dstf = adj[1]
    x_pad = jnp.pad(x, ((0, n_pad - n), (0, 0)))

    degp = _deg_kernel(n_pad, epw)(dstf).reshape(_NW * _NB, n_pad)

    degrow = pl.pallas_call(
        _tc_reduce_body,
        out_shape=jax.ShapeDtypeStruct((1, n_pad), jnp.float32),
    )(degp)
    deg_col = degrow.reshape(n_pad, 1)

    dinv, xs = pl.pallas_call(
        _tc_scale_body,
        out_shape=[
            jax.ShapeDtypeStruct((n_pad, 1), jnp.float32),
            jax.ShapeDtypeStruct((n_pad, nfeat), jnp.float32),
        ],
    )(deg_col, x_pad)

    y1 = _from_cb(_prop_kernel(n_pad, nchunk)(_to_cb(xs, n_pad), srcf, dstf),
                  n_pad)

    g = pl.pallas_call(
        _tc_mid_body,
        out_shape=jax.ShapeDtypeStruct((n_pad, nhid), jnp.float32),
    )(y1, dinv, W1, b1.reshape(1, -1), W2)

    y2 = _from_cb(_prop_kernel(n_pad, nchunk)(_to_cb(g, n_pad), srcf, dstf),
                  n_pad)

    out = pl.pallas_call(
        _tc_out_body,
        out_shape=jax.ShapeDtypeStruct((n_pad, nhid), jnp.float32),
    )(y2, dinv, b2.reshape(1, -1))

    return out[:n]
